# baseline (speedup 1.0000x reference)
import jax
import jax.numpy as jnp
from jax import lax
from jax.experimental import pallas as pl
from jax.experimental.pallas import tpu as pltpu
from jax.experimental.pallas import tpu_sc as plsc

NUM_VOCAB = 100000
MAXLEN = 200
NUM_HID = 128
BATCH = 4096
SEQ = 200

NC, NS, L = 2, 16, 16
NW = NC * NS
ROWS = BATCH * SEQ
CHUNK = 128
CPW = ROWS // NW // CHUNK
NCHUNKS = ROWS // CHUNK
HGRP = NUM_HID // L
NPAIR = CPW // 2
NITER = NPAIR - 2


def _body(x2, s_emb, pos_emb, out, i0, i1, i2, i3, i4, i5, B, posb,
          gi0, gi1, gi2, gi3, gi4, gi5, g0, g1, g2, g3, g4, g5,
          o0, o1, o2):
    wid = lax.axis_index("s") * NC + lax.axis_index("c")
    cb = wid * CPW

    pltpu.sync_copy(pos_emb, posb)

    idxs = (i0, i1, i2, i3, i4, i5)
    isems = (gi0, gi1, gi2, gi3, gi4, gi5)
    gsems = (g0, g1, g2, g3, g4, g5)
    osems = (o0, o1, o2)

    def start_idx(c, s):
        cc = jnp.minimum(cb + c, NCHUNKS - 1)
        pltpu.async_copy(x2.at[pl.ds(cc, 1)], idxs[s], isems[s])

    def wait_idx(s):
        pltpu.make_async_copy(x2.at[pl.ds(0, 1)], idxs[s], isems[s]).wait()

    def start_gather(c, s):
        pltpu.async_copy(s_emb.at[idxs[s].at[0]], B.at[s], gsems[s])

    def wait_gather(s):
        pltpu.make_async_copy(s_emb.at[idxs[s].at[0]], B.at[s], gsems[s]).wait()

    def start_scatter2(c, pp):
        pltpu.async_copy(B.at[pl.ds(2 * pp, 2)], out.at[pl.ds(cb + c, 2)],
                         osems[pp])

    def wait_scatter2(pp):
        pltpu.make_async_copy(B.at[pl.ds(2 * pp, 2)], out.at[pl.ds(0, 2)],
                              osems[pp]).wait()

    def add_pos(c, s):
        off = lax.rem(c * CHUNK, MAXLEN)
        m1 = jnp.minimum(MAXLEN - off, CHUNK)

        def add_lo(r, acc):
            for cg in range(HGRP):
                pv = posb[off + r, pl.ds(cg * L, L)]
                plsc.addupdate(B.at[s, r, pl.ds(cg * L, L)], pv)
            return acc

        def add_hi(r, acc):
            for cg in range(HGRP):
                pv = posb[off - MAXLEN + r, pl.ds(cg * L, L)]
                plsc.addupdate(B.at[s, r, pl.ds(cg * L, L)], pv)
            return acc

        lax.fori_loop(0, m1, add_lo, 0)
        lax.fori_loop(m1, CHUNK, add_hi, 0)

    for c in range(6):
        start_idx(c, c)
    for c in range(4):
        wait_idx(c)
        start_gather(c, c)

    def step3(t, carry):
        for P in range(3):
            k = 3 * t + P
            c0 = 2 * k
            s0, s1 = 2 * P, 2 * P + 1
            Pn = (P + 2) % 3
            sn0, sn1 = 2 * Pn, 2 * Pn + 1

            wait_gather(s0)
            start_idx(c0 + 6, s0)
            add_pos(c0, s0)
            wait_gather(s1)
            start_idx(c0 + 7, s1)
            add_pos(c0 + 1, s1)
            start_scatter2(c0, P)

            wait_idx(sn0)
            wait_idx(sn1)
            if P == 0:
                @pl.when(t > 0)
                def _():
                    wait_scatter2(Pn)
            else:
                wait_scatter2(Pn)

            start_gather(c0 + 4, sn0)
            start_gather(c0 + 5, sn1)
        return carry

    lax.fori_loop(0, NITER // 3, step3, 0)

    for k in range(3 * (NITER // 3), NPAIR):
        c0 = 2 * k
        P = k % 3
        s0, s1 = 2 * P, 2 * P + 1
        Pn = (P + 2) % 3
        sn0, sn1 = 2 * Pn, 2 * Pn + 1

        wait_gather(s0)
        add_pos(c0, s0)
        wait_gather(s1)
        add_pos(c0 + 1, s1)
        start_scatter2(c0, P)

        if k < NITER:
            start_idx(c0 + 6, s0)
            start_idx(c0 + 7, s1)
            wait_idx(sn0)
            wait_idx(sn1)
            wait_scatter2(Pn)
            start_gather(c0 + 4, sn0)
            start_gather(c0 + 5, sn1)

    for pp in range(3):
        wait_scatter2(pp)


@jax.jit
def _run(x2, s_emb, pos_emb):
    mesh = plsc.VectorSubcoreMesh(core_axis_name="c", subcore_axis_name="s")
    return pl.kernel(
        _body,
        out_type=jax.ShapeDtypeStruct((NCHUNKS, CHUNK, NUM_HID), jnp.float32),
        mesh=mesh,
        scratch_types=[
            pltpu.VMEM((1, CHUNK), jnp.int32),
            pltpu.VMEM((1, CHUNK), jnp.int32),
            pltpu.VMEM((1, CHUNK), jnp.int32),
            pltpu.VMEM((1, CHUNK), jnp.int32),
            pltpu.VMEM((1, CHUNK), jnp.int32),
            pltpu.VMEM((1, CHUNK), jnp.int32),
            pltpu.VMEM((6, CHUNK, NUM_HID), jnp.float32),
            pltpu.VMEM((MAXLEN, NUM_HID), jnp.float32),
            pltpu.SemaphoreType.DMA,
            pltpu.SemaphoreType.DMA,
            pltpu.SemaphoreType.DMA,
            pltpu.SemaphoreType.DMA,
            pltpu.SemaphoreType.DMA,
            pltpu.SemaphoreType.DMA,
            pltpu.SemaphoreType.DMA,
            pltpu.SemaphoreType.DMA,
            pltpu.SemaphoreType.DMA,
            pltpu.SemaphoreType.DMA,
            pltpu.SemaphoreType.DMA,
            pltpu.SemaphoreType.DMA,
            pltpu.SemaphoreType.DMA,
            pltpu.SemaphoreType.DMA,
            pltpu.SemaphoreType.DMA,
        ],
    )(x2, s_emb, pos_emb)


def kernel(x, s_emb, pos_emb):
    x2 = x.astype(jnp.int32).reshape(NCHUNKS, CHUNK)
    out = _run(x2, s_emb, pos_emb)
    return out.reshape(BATCH, SEQ, NUM_HID)


# speedup vs baseline: 2.6073x; 2.6073x over previous
import jax
import jax.numpy as jnp
from jax import lax
from jax.experimental import pallas as pl
from jax.experimental.pallas import tpu as pltpu
from jax.experimental.pallas import tpu_sc as plsc

NUM_VOCAB = 100000
MAXLEN = 200
NUM_HID = 128
BATCH = 4096
SEQ = 200

NC, NS, L = 2, 16, 16
NW = NC * NS
ROWS = BATCH * SEQ
SEQ_PER_W = BATCH // NW
HALF = SEQ // 2
HGRP = NUM_HID // L
NITER = (SEQ_PER_W - 2) // 3


def _body(x2, s_emb, pos_emb, out, i0, i1, i2, b0, b1, b2, posb,
          gi0, gi1, gi2, g0, g1, g2, o0, o1, o2):
    wid = lax.axis_index("s") * NC + lax.axis_index("c")
    cb = wid * SEQ_PER_W

    pltpu.sync_copy(pos_emb, posb)

    idxs = (i0, i1, i2)
    bufs = (b0, b1, b2)
    isems = (gi0, gi1, gi2)
    gsems = (g0, g1, g2)
    osems = (o0, o1, o2)
    last = SEQ_PER_W - 1

    def start_idx(c, p):
        cc = jnp.minimum(cb + c, NW * SEQ_PER_W - 1)
        pltpu.async_copy(x2.at[pl.ds(2 * cc, 2)], idxs[p], isems[p])

    def wait_idx(p):
        pltpu.make_async_copy(x2.at[pl.ds(0, 2)], idxs[p], isems[p]).wait()

    def start_gather(c, p):
        buf, sem = bufs[p], gsems[p]
        pltpu.async_copy(s_emb.at[idxs[p].at[0]], buf.at[pl.ds(0, HALF)], sem)
        pltpu.async_copy(s_emb.at[idxs[p].at[1]], buf.at[pl.ds(HALF, HALF)], sem)

    def wait_gather(p):
        buf, sem = bufs[p], gsems[p]
        pltpu.make_async_copy(s_emb.at[idxs[p].at[0]], buf.at[pl.ds(0, HALF)], sem).wait()
        pltpu.make_async_copy(s_emb.at[idxs[p].at[0]], buf.at[pl.ds(HALF, HALF)], sem).wait()

    def start_scatter(c, p):
        pltpu.async_copy(bufs[p], out.at[pl.ds((cb + c) * SEQ, SEQ)], osems[p])

    def wait_scatter(p):
        pltpu.make_async_copy(bufs[p], out.at[pl.ds(0, SEQ)], osems[p]).wait()

    def add_pos(p):
        buf = bufs[p]

        def add_row(r, acc):
            for cg in range(HGRP):
                pv = posb[r, pl.ds(cg * L, L)]
                plsc.addupdate(buf.at[r, pl.ds(cg * L, L)], pv)
            return acc

        lax.fori_loop(0, SEQ, add_row, 0, unroll=2)

    for c in range(3):
        start_idx(c, c)
    for c in range(2):
        wait_idx(c)
        start_gather(c, c)

    def step(it, carry):
        c0 = 3 * it
        for j in range(3):
            p = j
            pn = (j + 2) % 3
            wait_gather(p)
            start_idx(c0 + j + 3, p)
            add_pos(p)
            start_scatter(c0 + j, p)
            wait_idx(pn)
            if j == 0:
                @pl.when(it > 0)
                def _():
                    wait_scatter(pn)
            else:
                wait_scatter(pn)
            start_gather(c0 + j + 2, pn)
        return carry

    lax.fori_loop(0, NITER, step, 0)

    for c, p in ((SEQ_PER_W - 2, 0), (SEQ_PER_W - 1, 1)):
        wait_gather(p)
        add_pos(p)
        start_scatter(c, p)
    for p in (2, 0, 1):
        wait_scatter(p)


@jax.jit
def _run(x2, s_emb, pos_emb):
    mesh = plsc.VectorSubcoreMesh(core_axis_name="c", subcore_axis_name="s")
    return pl.kernel(
        _body,
        out_type=jax.ShapeDtypeStruct((ROWS, NUM_HID), jnp.float32),
        mesh=mesh,
        scratch_types=[
            pltpu.VMEM((2, HALF), jnp.int32),
            pltpu.VMEM((2, HALF), jnp.int32),
            pltpu.VMEM((2, HALF), jnp.int32),
            pltpu.VMEM((SEQ, NUM_HID), jnp.float32),
            pltpu.VMEM((SEQ, NUM_HID), jnp.float32),
            pltpu.VMEM((SEQ, NUM_HID), jnp.float32),
            pltpu.VMEM((MAXLEN, NUM_HID), jnp.float32),
            pltpu.SemaphoreType.DMA,
            pltpu.SemaphoreType.DMA,
            pltpu.SemaphoreType.DMA,
            pltpu.SemaphoreType.DMA,
            pltpu.SemaphoreType.DMA,
            pltpu.SemaphoreType.DMA,
            pltpu.SemaphoreType.DMA,
            pltpu.SemaphoreType.DMA,
            pltpu.SemaphoreType.DMA,
        ],
    )(x2, s_emb, pos_emb)


def kernel(x, s_emb, pos_emb):
    x2 = x.astype(jnp.int32).reshape(BATCH * 2, SEQ // 2)
    out = _run(x2, s_emb, pos_emb)
    return out.reshape(BATCH, SEQ, NUM_HID)
